# baseline (speedup 1.0000x reference)
import jax
import jax.numpy as jnp
from jax.experimental import pallas as pl
from jax.experimental.pallas import tpu as pltpu

_PAIR_I = (0, 0, 0, 1, 1, 2)
_PAIR_J = (1, 2, 3, 2, 3, 3)
_N = 8192
_ROWS = 1024
_CHUNKS = 4
_MXU_COLS = 5120
_EPS = 1e-12


def _loss_body(pa_ref, pb_ref, a_ref, b_ref, out_ref, baug_ref):
    p = pl.program_id(0)
    r = pl.program_id(1)
    np_ = pl.num_programs(0)
    nr = pl.num_programs(1)

    @pl.when((p == 0) & (r == 0))
    def _init():
        out_ref[0, 0] = 0.0

    @pl.when(r == 0)
    def _build_b():
        bx = b_ref[0, 0:1, :]
        by = b_ref[0, 1:2, :]
        bz = b_ref[0, 2:3, :]
        wx = pb_ref[p, 0] * bx + pb_ref[p, 1] * by + pb_ref[p, 2] * bz + pb_ref[p, 3]
        wy = pb_ref[p, 4] * bx + pb_ref[p, 5] * by + pb_ref[p, 6] * bz + pb_ref[p, 7]
        wz = pb_ref[p, 8] * bx + pb_ref[p, 9] * by + pb_ref[p, 10] * bz + pb_ref[p, 11]
        baug_ref[0:1, :] = -2.0 * wx
        baug_ref[1:2, :] = -2.0 * wy
        baug_ref[2:3, :] = -2.0 * wz
        baug_ref[3:4, :] = wx * wx + wy * wy + wz * wz

    ax = a_ref[0, 0:1, :]
    ay = a_ref[0, 1:2, :]
    az = a_ref[0, 2:3, :]
    wax = pa_ref[p, 0] * ax + pa_ref[p, 1] * ay + pa_ref[p, 2] * az + pa_ref[p, 3]
    way = pa_ref[p, 4] * ax + pa_ref[p, 5] * ay + pa_ref[p, 6] * az + pa_ref[p, 7]
    waz = pa_ref[p, 8] * ax + pa_ref[p, 9] * ay + pa_ref[p, 10] * az + pa_ref[p, 11]
    a2 = wax * wax + way * way + waz * waz
    aaug = jnp.concatenate([wax, way, waz, jnp.ones_like(wax)], axis=0)

    chunk = _MXU_COLS // _CHUNKS
    m = None
    for k in range(_CHUNKS):
        h = jax.lax.dot_general(
            aaug, baug_ref[:, k * chunk:(k + 1) * chunk],
            dimension_numbers=(((0,), (0,)), ((), ())),
            preferred_element_type=jnp.float32,
        )
        mk = jnp.min(h, axis=1, keepdims=True)
        m = mk if m is None else jnp.minimum(m, mk)

    waxc = jnp.transpose(wax)
    wayc = jnp.transpose(way)
    wazc = jnp.transpose(waz)
    bxd = baug_ref[0:1, _MXU_COLS:]
    byd = baug_ref[1:2, _MXU_COLS:]
    bzd = baug_ref[2:3, _MXU_COLS:]
    b2v = baug_ref[3:4, _MXU_COLS:]
    hv = (waxc * bxd + wayc * byd) + (wazc * bzd + b2v)
    m = jnp.minimum(m, jnp.min(hv, axis=1, keepdims=True))
    d = jnp.sqrt(jnp.maximum(jnp.transpose(m) + a2, _EPS))
    out_ref[0, 0] += jnp.sum(d)

    @pl.when((p == np_ - 1) & (r == nr - 1))
    def _finish():
        out_ref[0, 0] = out_ref[0, 0] * (1.0 / (6.0 * _N))


def kernel(point_clouds, camera_poses):
    idx_i = jnp.array(_PAIR_I)
    idx_j = jnp.array(_PAIR_J)
    pc_t = jnp.transpose(point_clouds, (0, 2, 1))
    a_in = pc_t[idx_i]
    b_in = pc_t[idx_j]
    pose_rows = camera_poses[:, :3, :].reshape(4, 12)
    pa = pose_rows[idx_i]
    pb = pose_rows[idx_j]

    nr = _N // _ROWS
    out = pl.pallas_call(
        _loss_body,
        grid=(len(_PAIR_I), nr),
        in_specs=[
            pl.BlockSpec(memory_space=pltpu.SMEM),
            pl.BlockSpec(memory_space=pltpu.SMEM),
            pl.BlockSpec((1, 3, _ROWS), lambda p, r: (p, 0, r)),
            pl.BlockSpec((1, 3, _N), lambda p, r: (p, 0, 0)),
        ],
        out_specs=pl.BlockSpec(memory_space=pltpu.SMEM),
        out_shape=jax.ShapeDtypeStruct((1, 1), jnp.float32),
        scratch_shapes=[pltpu.VMEM((4, _N), jnp.float32)],
    )(pa, pb, a_in, b_in)
    return out[0, 0]


# speedup vs baseline: 1.4432x; 1.4432x over previous
import jax
import jax.numpy as jnp
from jax.experimental import pallas as pl
from jax.experimental.pallas import tpu as pltpu

_PAIR_I = (0, 0, 0, 1, 1, 2)
_PAIR_J = (1, 2, 3, 2, 3, 3)
_N = 8192
_ROWS = 1024
_CHUNKS = 4
_MXU_COLS = _N
_EPS = 1e-12


def _loss_body(pa_ref, pb_ref, a_ref, b_ref, out_ref, baug_ref):
    p = pl.program_id(0)
    r = pl.program_id(1)
    np_ = pl.num_programs(0)
    nr = pl.num_programs(1)

    @pl.when((p == 0) & (r == 0))
    def _init():
        out_ref[0, 0] = 0.0

    @pl.when(r == 0)
    def _build_b():
        bx = b_ref[0, 0:1, :]
        by = b_ref[0, 1:2, :]
        bz = b_ref[0, 2:3, :]
        wx = pb_ref[p, 0] * bx + pb_ref[p, 1] * by + pb_ref[p, 2] * bz + pb_ref[p, 3]
        wy = pb_ref[p, 4] * bx + pb_ref[p, 5] * by + pb_ref[p, 6] * bz + pb_ref[p, 7]
        wz = pb_ref[p, 8] * bx + pb_ref[p, 9] * by + pb_ref[p, 10] * bz + pb_ref[p, 11]
        baug_ref[0:1, :] = -2.0 * wx
        baug_ref[1:2, :] = -2.0 * wy
        baug_ref[2:3, :] = -2.0 * wz
        baug_ref[3:4, :] = wx * wx + wy * wy + wz * wz

    ax = a_ref[0, 0:1, :]
    ay = a_ref[0, 1:2, :]
    az = a_ref[0, 2:3, :]
    wax = pa_ref[p, 0] * ax + pa_ref[p, 1] * ay + pa_ref[p, 2] * az + pa_ref[p, 3]
    way = pa_ref[p, 4] * ax + pa_ref[p, 5] * ay + pa_ref[p, 6] * az + pa_ref[p, 7]
    waz = pa_ref[p, 8] * ax + pa_ref[p, 9] * ay + pa_ref[p, 10] * az + pa_ref[p, 11]
    a2 = wax * wax + way * way + waz * waz
    aaug = jnp.concatenate([wax, way, waz, jnp.ones_like(wax)], axis=0)

    chunk = _N // _CHUNKS
    m = None
    for k in range(_CHUNKS):
        ht = jax.lax.dot_general(
            baug_ref[:, k * chunk:(k + 1) * chunk], aaug,
            dimension_numbers=(((0,), (0,)), ((), ())),
            preferred_element_type=jnp.float32,
        )
        mk = jnp.min(ht, axis=0, keepdims=True)
        m = mk if m is None else jnp.minimum(m, mk)

    d = jnp.sqrt(jnp.maximum(m + a2, _EPS))
    out_ref[0, 0] += jnp.sum(d)

    @pl.when((p == np_ - 1) & (r == nr - 1))
    def _finish():
        out_ref[0, 0] = out_ref[0, 0] * (1.0 / (6.0 * _N))


def kernel(point_clouds, camera_poses):
    idx_i = jnp.array(_PAIR_I)
    idx_j = jnp.array(_PAIR_J)
    pc_t = jnp.transpose(point_clouds, (0, 2, 1))
    a_in = pc_t[idx_i]
    b_in = pc_t[idx_j]
    pose_rows = camera_poses[:, :3, :].reshape(4, 12)
    pa = pose_rows[idx_i]
    pb = pose_rows[idx_j]

    nr = _N // _ROWS
    out = pl.pallas_call(
        _loss_body,
        grid=(len(_PAIR_I), nr),
        in_specs=[
            pl.BlockSpec(memory_space=pltpu.SMEM),
            pl.BlockSpec(memory_space=pltpu.SMEM),
            pl.BlockSpec((1, 3, _ROWS), lambda p, r: (p, 0, r)),
            pl.BlockSpec((1, 3, _N), lambda p, r: (p, 0, 0)),
        ],
        out_specs=pl.BlockSpec(memory_space=pltpu.SMEM),
        out_shape=jax.ShapeDtypeStruct((1, 1), jnp.float32),
        scratch_shapes=[pltpu.VMEM((4, _N), jnp.float32)],
    )(pa, pb, a_in, b_in)
    return out[0, 0]
